# Initial kernel scaffold; baseline (speedup 1.0000x reference)
import jax
import jax.numpy as jnp
from jax.experimental import pallas as pl


def kernel(x_movement, edge_src, edge_dst, phase_index):
    raise NotImplementedError("write your pallas kernel here")


# speedup vs baseline: 7.2764x; 7.2764x over previous
import functools

import jax
import jax.numpy as jnp
from jax import lax
from jax.experimental import pallas as pl
from jax.experimental.pallas import tpu as pltpu
from jax.experimental.pallas import tpu_sc as plsc

N_MOV = 100000
N_EDGE = 1600000
N_PHASE = 50000
N_INTER = 12500

NC, NS = 2, 16
NW = NC * NS
CE = 12800
NCH = N_EDGE // CE
CPT = -(-NCH // NW)

INT_MAX = 2147483647


def _diff_body(xt_ref, o_ref):
    o_ref[...] = xt_ref[0, :] - xt_ref[1, :]


def _diff(xt):
    return pl.pallas_call(
        _diff_body,
        out_shape=jax.ShapeDtypeStruct((N_MOV,), jnp.float32),
    )(xt)


_sc_mesh = plsc.VectorSubcoreMesh(core_axis_name="c", subcore_axis_name="s")


@functools.partial(
    pl.kernel,
    mesh=_sc_mesh,
    out_type=jax.ShapeDtypeStruct((NC, N_PHASE), jnp.float32),
    scratch_types=[
        pltpu.VMEM((CE,), jnp.int32),
        pltpu.VMEM((CE,), jnp.int32),
        pltpu.VMEM((CE,), jnp.float32),
        pltpu.VMEM_SHARED((N_PHASE,), jnp.float32),
        pltpu.SemaphoreType.DMA,
    ],
)
def _seg_sum(diff_hbm, src_hbm, dst_hbm, zeros_hbm, out_hbm,
             src_v, dst_v, vals_v, acc, sem):
    c = lax.axis_index("c")
    s = lax.axis_index("s")
    wid = c * NS + s

    @pl.when(s == 0)
    def _():
        pltpu.sync_copy(zeros_hbm, acc)

    plsc.subcore_barrier()

    for i in range(CPT):
        ch = i * NW + wid

        @pl.when(ch < NCH)
        def _():
            sl = pl.ds(ch * CE, CE)
            pltpu.sync_copy(src_hbm.at[sl], src_v)
            pltpu.sync_copy(dst_hbm.at[sl], dst_v)
            pltpu.async_copy(diff_hbm.at[src_v], vals_v, sem).wait()
            pltpu.sync_copy(vals_v, acc.at[dst_v], add=True)

    plsc.subcore_barrier()

    @pl.when(s == 0)
    def _():
        pltpu.sync_copy(acc, out_hbm.at[c])


def _scan_body(part_ref, pi_ref, o_ref):
    p = part_ref[0, :] + part_ref[1, :]
    k = pi_ref[...]
    NEG = jnp.float32(-3.0e38)

    def shift_fwd(v, d, pad):
        return jnp.concatenate([jnp.full((d,), pad, v.dtype), v[:-d]])

    def shift_bwd(v, d, pad):
        return jnp.concatenate([v[d:], jnp.full((d,), pad, v.dtype)])

    def seg_scan(v, combine, pad, bwd=False):
        sh = shift_bwd if bwd else shift_fwd
        d = 1
        while d < N_PHASE:
            pv = sh(v, d, pad)
            pk = sh(k, d, jnp.int32(-1))
            v = jnp.where(pk == k, combine(v, pv), v)
            d *= 2
        return v

    segmax = jnp.maximum(
        seg_scan(p, jnp.maximum, NEG),
        seg_scan(p, jnp.maximum, NEG, bwd=True),
    )
    local = seg_scan(jnp.ones((N_PHASE,), jnp.int32), jnp.add, jnp.int32(0)) - 1
    cand = jnp.where(p == segmax, local, jnp.int32(N_PHASE))
    imax = jnp.int32(INT_MAX)
    o_ref[...] = jnp.minimum(
        seg_scan(cand, jnp.minimum, imax),
        seg_scan(cand, jnp.minimum, imax, bwd=True),
    )


def _seg_argmax(partials, phase_index):
    return pl.pallas_call(
        _scan_body,
        out_shape=jax.ShapeDtypeStruct((N_PHASE,), jnp.int32),
    )(partials, phase_index)


@functools.partial(
    pl.kernel,
    mesh=_sc_mesh,
    out_type=jax.ShapeDtypeStruct((N_INTER,), jnp.int32),
    scratch_types=[
        pltpu.VMEM((N_PHASE,), jnp.int32),
        pltpu.VMEM((N_PHASE,), jnp.int32),
        pltpu.VMEM((N_INTER,), jnp.int32),
    ],
)
def _compress(pi_hbm, sm_hbm, init_hbm, out_hbm, pi_v, sm_v, init_v):
    c = lax.axis_index("c")
    s = lax.axis_index("s")

    @pl.when((c == 0) & (s == 0))
    def _():
        pltpu.sync_copy(init_hbm, init_v)
        pltpu.sync_copy(init_v, out_hbm)
        pltpu.sync_copy(pi_hbm, pi_v)
        pltpu.sync_copy(sm_hbm, sm_v)
        pltpu.sync_copy(sm_v, out_hbm.at[pi_v])


@jax.jit
def kernel(x_movement, edge_src, edge_dst, phase_index):
    diff = _diff(x_movement.T)
    partials = _seg_sum(
        diff, edge_src, edge_dst, jnp.zeros((N_PHASE,), jnp.float32)
    )
    segmin = _seg_argmax(partials, phase_index)
    return _compress(
        phase_index, segmin, jnp.full((N_INTER,), INT_MAX, jnp.int32)
    )
